# Initial kernel scaffold; baseline (speedup 1.0000x reference)
import jax
import jax.numpy as jnp
from jax.experimental import pallas as pl


def kernel(scores, labels):
    raise NotImplementedError("write your pallas kernel here")


# speedup vs baseline: 4.5197x; 4.5197x over previous
import functools

import jax
import jax.numpy as jnp
from jax import lax
from jax.experimental import pallas as pl
from jax.experimental.pallas import tpu as pltpu


def _loss_body(s_ref, lab_ref, out_ref):
    B, n = s_ref.shape
    half = n // 2
    s = s_ref[:]
    lab = lab_ref[:]

    li = lab[:, None, :]
    lj = lab[:, :, None]
    ii = lax.broadcasted_iota(jnp.int32, (B, n, n), 2)
    jj = lax.broadcasted_iota(jnp.int32, (B, n, n), 1)
    beats = (lj > li) | ((lj == li) & (jj < ii))
    rank = jnp.sum(beats.astype(jnp.int32), axis=1)

    kk = lax.broadcasted_iota(jnp.int32, (B, n, n), 2)
    onehot = (rank[:, :, None] == kk).astype(jnp.float32)
    os_ = jnp.sum(s[:, :, None] * onehot, axis=1)

    e = jnp.exp(os_)
    einv = jnp.exp(-os_)

    ju = lax.broadcasted_iota(jnp.int32, (B, half, n), 2)
    jj2 = lax.broadcasted_iota(jnp.int32, (B, half, n), 1)
    maskA = ((ju >= jj2) & (ju <= half)).astype(jnp.float32)
    maskC = (ju >= (n - half - 1 + jj2)).astype(jnp.float32)
    A = jnp.sum(e[:, None, :] * maskA, axis=2)
    C = jnp.sum(einv[:, None, :] * maskC, axis=2)

    jvec = lax.broadcasted_iota(jnp.int32, (B, half), 1).astype(jnp.float32)
    cnt = jnp.maximum(0.0, 2.0 - jvec)
    denom = A * C - cnt
    logden = jnp.sum(jnp.log(denom))

    sgnsum = jnp.sum(s * jnp.where(rank < half, 1.0, -1.0))

    out_ref[...] = jnp.reshape(-(sgnsum - logden) / B, (1, 1))


@jax.jit
def kernel(scores, labels):
    B, n, _ = scores.shape
    s = scores[..., 0]
    if n % 2 != 0:
        s = s[:, :-1]
        labels = labels[:, :-1]
        n -= 1
    out = pl.pallas_call(
        _loss_body,
        out_shape=jax.ShapeDtypeStruct((1, 1), jnp.float32),
    )(s, labels)
    return out[0, 0]
